# baseline (speedup 1.0000x reference)
import jax
import jax.numpy as jnp
from jax.experimental import pallas as pl
from jax.experimental.pallas import tpu as pltpu

NUM_VOCAB = 100000
OP_VOCAB = 16
EMB = 128
CLS = 100000
K_SPARSE = 64

BLK = 5000
NBLK = CLS // BLK
C = 1000
R = CLS // C
SUB = BLK // C


def _lane_cumsum(x, lanes):
    k = 1
    while k < C:
        sh = pltpu.roll(x, k, axis=1)
        x = x + jnp.where(lanes >= k, sh, 0)
        k *= 2
    return x


def _row_cumsum(x, rows):
    k = 1
    while k < R:
        sh = pltpu.roll(x, k, axis=0)
        x = x + jnp.where(rows >= k, sh, 0)
        k *= 2
    return x


def _body(a_s, op_s, b_s,
          arow, oprow, brow, w, bias,
          out_sp, out_idx, out_val,
          pv, wk, ms):
    i = pl.program_id(0)
    c = jnp.concatenate([arow[0, 0, :], oprow[0, 0, :], brow[0, 0, :]])
    c2 = c.reshape(1, 3 * EMB)
    for s in range(SUB):
        proj = jax.lax.dot_general(
            c2, w[C * s:C * (s + 1), :], (((1,), (1,)), ((), ())),
            preferred_element_type=jnp.float32)
        proj = proj + bias[0, 0, C * s:C * (s + 1)].reshape(1, C)
        key = jax.lax.bitcast_convert_type(jnp.abs(proj), jnp.int32)
        pv[pl.ds(SUB * i + s, 1), :] = proj
        wk[pl.ds(SUB * i + s, 1), :] = key

    @pl.when(i == NBLK - 1)
    def _finalize():
        rows = jax.lax.broadcasted_iota(jnp.int32, (R, 1), 0)
        lanes = jax.lax.broadcasted_iota(jnp.int32, (R, C), 1)
        lanes_c = jax.lax.broadcasted_iota(jnp.int32, (1, C), 1)
        lanes_k = jax.lax.broadcasted_iota(jnp.int32, (1, K_SPARSE), 1)

        key = wk[...]

        def dstep(j, prefix):
            cand = prefix + jax.lax.shift_left(1, 30 - j)
            cnt = jnp.sum((key >= cand).astype(jnp.int32))
            return jnp.where(cnt >= K_SPARSE, cand, prefix)

        T = jax.lax.fori_loop(0, 31, dstep, jnp.int32(0))

        strict = key > T
        tie = key == T
        need = K_SPARSE - jnp.sum(strict.astype(jnp.int32))
        tcum = _lane_cumsum(tie.astype(jnp.int32), lanes)
        trt = tcum[:, C - 1:C]
        toff = _row_cumsum(trt, rows) - trt
        sel = strict | (tie & ((tcum + toff) <= need))

        scum = _lane_cumsum(sel.astype(jnp.int32), lanes)
        srt = scum[:, C - 1:C]
        sinc = _row_cumsum(srt, rows)
        soff = sinc - srt
        gcum = scum + soff
        msel = jnp.where(sel, gcum, 0)
        ms[...] = msel

        row_k = jnp.sum((soff <= lanes_k).astype(jnp.int32), axis=0,
                        keepdims=True) - 1

        def slot(k, carry):
            fidx, fval = carry
            r = jnp.sum(jnp.where(lanes_k == k, row_k, 0))
            mrow = ms[pl.ds(r, 1), :]
            prow = pv[pl.ds(r, 1), :]
            hit = mrow == k + 1
            col = jnp.sum(jnp.where(hit, lanes_c, 0))
            v = jnp.sum(jnp.where(hit, prow, 0.0))
            flat = r * C + col
            fidx = jnp.where(lanes_k == k, flat, fidx)
            fval = jnp.where(lanes_k == k, v, fval)
            return fidx, fval

        fidx, fval = jax.lax.fori_loop(
            0, K_SPARSE, slot,
            (jnp.zeros((1, K_SPARSE), jnp.int32),
             jnp.zeros((1, K_SPARSE), jnp.float32)))

        out_idx[...] = fidx
        out_val[...] = fval
        sp = jnp.where(msel > 0, pv[...], 0.0)
        out_sp[...] = sp.reshape(R, 1, C)


@jax.jit
def kernel(a, op_idx, b, num_emb, op_emb, W, bias):
    a1 = jnp.asarray(a, jnp.int32).reshape(1)
    o1 = jnp.asarray(op_idx, jnp.int32).reshape(1)
    b1 = jnp.asarray(b, jnp.int32).reshape(1)
    bias3 = bias.reshape(NBLK, 1, BLK)
    ne3 = num_emb.reshape(NUM_VOCAB, 1, EMB)
    oe3 = op_emb.reshape(OP_VOCAB, 1, EMB)

    grid_spec = pltpu.PrefetchScalarGridSpec(
        num_scalar_prefetch=3,
        grid=(NBLK,),
        in_specs=[
            pl.BlockSpec((1, 1, EMB), lambda i, a_s, o_s, b_s: (a_s[0], 0, 0)),
            pl.BlockSpec((1, 1, EMB), lambda i, a_s, o_s, b_s: (o_s[0], 0, 0)),
            pl.BlockSpec((1, 1, EMB), lambda i, a_s, o_s, b_s: (b_s[0], 0, 0)),
            pl.BlockSpec((BLK, 3 * EMB), lambda i, a_s, o_s, b_s: (i, 0)),
            pl.BlockSpec((1, 1, BLK), lambda i, a_s, o_s, b_s: (i, 0, 0)),
        ],
        out_specs=[
            pl.BlockSpec((R, 1, C), lambda i, a_s, o_s, b_s: (0, 0, 0)),
            pl.BlockSpec((1, K_SPARSE), lambda i, a_s, o_s, b_s: (0, 0)),
            pl.BlockSpec((1, K_SPARSE), lambda i, a_s, o_s, b_s: (0, 0)),
        ],
        scratch_shapes=[
            pltpu.VMEM((R, C), jnp.float32),
            pltpu.VMEM((R, C), jnp.int32),
            pltpu.VMEM((R, C), jnp.int32),
        ],
    )
    sp, sidx, sval = pl.pallas_call(
        _body,
        grid_spec=grid_spec,
        out_shape=[
            jax.ShapeDtypeStruct((R, 1, C), jnp.float32),
            jax.ShapeDtypeStruct((1, K_SPARSE), jnp.int32),
            jax.ShapeDtypeStruct((1, K_SPARSE), jnp.float32),
        ],
    )(a1, o1, b1, ne3, oe3, ne3, W, bias3)
    return sp.reshape(CLS), sidx.reshape(K_SPARSE), sval.reshape(K_SPARSE)


# speedup vs baseline: 1.1329x; 1.1329x over previous
import jax
import jax.numpy as jnp
from jax.experimental import pallas as pl
from jax.experimental.pallas import tpu as pltpu

NUM_VOCAB = 100000
OP_VOCAB = 16
EMB = 128
CLS = 100000
K_SPARSE = 64

BLK = 5000
NBLK = CLS // BLK
C = 1000
R = CLS // C
SUB = BLK // C


def _lane_cumsum(x, lanes):
    k = 1
    while k < C:
        sh = pltpu.roll(x, k, axis=1)
        x = x + jnp.where(lanes >= k, sh, 0)
        k *= 2
    return x


def _row_cumsum(x, rows):
    k = 1
    while k < R:
        sh = pltpu.roll(x, k, axis=0)
        x = x + jnp.where(rows >= k, sh, 0)
        k *= 2
    return x


def _body(a_s, op_s, b_s,
          arow, oprow, brow, w, bias,
          out_sp, out_idx, out_val,
          pv, wk, ms):
    i = pl.program_id(0)
    c = jnp.concatenate([arow[0, 0, :], oprow[0, 0, :], brow[0, 0, :]])
    c2 = c.reshape(1, 3 * EMB)
    for s in range(SUB):
        proj = jax.lax.dot_general(
            c2, w[C * s:C * (s + 1), :], (((1,), (1,)), ((), ())),
            preferred_element_type=jnp.float32)
        proj = proj + bias[0, 0, C * s:C * (s + 1)].reshape(1, C)
        key = jax.lax.bitcast_convert_type(jnp.abs(proj), jnp.int32)
        pv[pl.ds(SUB * i + s, 1), :] = proj
        wk[pl.ds(SUB * i + s, 1), :] = key

    @pl.when(i == NBLK - 1)
    def _finalize():
        rows = jax.lax.broadcasted_iota(jnp.int32, (R, 1), 0)
        lanes = jax.lax.broadcasted_iota(jnp.int32, (R, C), 1)
        lanes_c = jax.lax.broadcasted_iota(jnp.int32, (1, C), 1)
        lanes_k = jax.lax.broadcasted_iota(jnp.int32, (1, K_SPARSE), 1)

        key = wk[...]

        prefix = jnp.int32(0)
        for j in range(31):
            cand = prefix + (1 << (30 - j))
            cnt = jnp.sum((key >= cand).astype(jnp.int32))
            prefix = jnp.where(cnt >= K_SPARSE, cand, prefix)
        T = prefix

        strict = key > T
        tie = key == T
        need = K_SPARSE - jnp.sum(strict.astype(jnp.int32))
        tcum = _lane_cumsum(tie.astype(jnp.int32), lanes)
        trt = tcum[:, C - 1:C]
        toff = _row_cumsum(trt, rows) - trt
        sel = strict | (tie & ((tcum + toff) <= need))

        scum = _lane_cumsum(sel.astype(jnp.int32), lanes)
        srt = scum[:, C - 1:C]
        sinc = _row_cumsum(srt, rows)
        soff = sinc - srt
        gcum = scum + soff
        msel = jnp.where(sel, gcum, 0)
        ms[...] = msel

        row_k = jnp.sum((soff <= lanes_k).astype(jnp.int32), axis=0,
                        keepdims=True) - 1

        fidx = jnp.zeros((1, K_SPARSE), jnp.int32)
        fval = jnp.zeros((1, K_SPARSE), jnp.float32)
        for k in range(K_SPARSE):
            r = jnp.sum(jnp.where(lanes_k == k, row_k, 0))
            mrow = ms[pl.ds(r, 1), :]
            prow = pv[pl.ds(r, 1), :]
            hit = mrow == k + 1
            col = jnp.sum(jnp.where(hit, lanes_c, 0))
            v = jnp.sum(jnp.where(hit, prow, 0.0))
            flat = r * C + col
            fidx = jnp.where(lanes_k == k, flat, fidx)
            fval = jnp.where(lanes_k == k, v, fval)

        out_idx[...] = fidx
        out_val[...] = fval
        sp = jnp.where(msel > 0, pv[...], 0.0)
        out_sp[...] = sp.reshape(R, 1, C)


@jax.jit
def kernel(a, op_idx, b, num_emb, op_emb, W, bias):
    a1 = jnp.asarray(a, jnp.int32).reshape(1)
    o1 = jnp.asarray(op_idx, jnp.int32).reshape(1)
    b1 = jnp.asarray(b, jnp.int32).reshape(1)
    bias3 = bias.reshape(NBLK, 1, BLK)
    ne3 = num_emb.reshape(NUM_VOCAB, 1, EMB)
    oe3 = op_emb.reshape(OP_VOCAB, 1, EMB)

    grid_spec = pltpu.PrefetchScalarGridSpec(
        num_scalar_prefetch=3,
        grid=(NBLK,),
        in_specs=[
            pl.BlockSpec((1, 1, EMB), lambda i, a_s, o_s, b_s: (a_s[0], 0, 0)),
            pl.BlockSpec((1, 1, EMB), lambda i, a_s, o_s, b_s: (o_s[0], 0, 0)),
            pl.BlockSpec((1, 1, EMB), lambda i, a_s, o_s, b_s: (b_s[0], 0, 0)),
            pl.BlockSpec((BLK, 3 * EMB), lambda i, a_s, o_s, b_s: (i, 0)),
            pl.BlockSpec((1, 1, BLK), lambda i, a_s, o_s, b_s: (i, 0, 0)),
        ],
        out_specs=[
            pl.BlockSpec((R, 1, C), lambda i, a_s, o_s, b_s: (0, 0, 0)),
            pl.BlockSpec((1, K_SPARSE), lambda i, a_s, o_s, b_s: (0, 0)),
            pl.BlockSpec((1, K_SPARSE), lambda i, a_s, o_s, b_s: (0, 0)),
        ],
        scratch_shapes=[
            pltpu.VMEM((R, C), jnp.float32),
            pltpu.VMEM((R, C), jnp.int32),
            pltpu.VMEM((R, C), jnp.int32),
        ],
    )
    sp, sidx, sval = pl.pallas_call(
        _body,
        grid_spec=grid_spec,
        out_shape=[
            jax.ShapeDtypeStruct((R, 1, C), jnp.float32),
            jax.ShapeDtypeStruct((1, K_SPARSE), jnp.int32),
            jax.ShapeDtypeStruct((1, K_SPARSE), jnp.float32),
        ],
    )(a1, o1, b1, ne3, oe3, ne3, W, bias3)
    return sp.reshape(CLS), sidx.reshape(K_SPARSE), sval.reshape(K_SPARSE)
